# baseline (speedup 1.0000x reference)
import jax
import jax.numpy as jnp
from jax import lax
from jax.experimental import pallas as pl
from jax.experimental.pallas import tpu as pltpu
from jax.experimental.pallas import tpu_sc as plsc

_N = 10000
_E = 320000
_D = 128
_NC, _NS = 2, 16
_RPT = 640
_NP = _NS * _RPT
_EB = 128
_TB = 80
_TB_PAIR = 160
_TB0 = 80
_TB1 = _TB_PAIR - _TB0
_TB_C = 40
_EPAD = _NS * _TB_PAIR * _EB
_RB = 1024
_ZB = 128


def _deg_body(dst_hbm, ones_hbm, out_hbm, acc_s, idx_v, buf_v, sem):
    cid = lax.axis_index("c")
    sid = lax.axis_index("s")
    r0 = sid * _RPT
    pltpu.sync_copy(dst_hbm.at[sid, pl.ds(cid * _TB, _TB)], idx_v)
    pltpu.sync_copy(ones_hbm.at[0], buf_v)
    for k in range(_RPT // _ZB):
        pltpu.sync_copy(buf_v, acc_s.at[pl.ds(r0 + k * _ZB, _ZB)])
    pltpu.sync_copy(ones_hbm.at[1], buf_v)
    plsc.subcore_barrier()

    def body(m, carry):
        for k in range(8):
            pltpu.async_copy(buf_v, acc_s.at[idx_v.at[m * 8 + k]], sem,
                             add=True)
        for k in range(8):
            pltpu.make_async_copy(buf_v, acc_s.at[idx_v.at[m * 8 + k]],
                                  sem).wait()
        return carry

    lax.fori_loop(0, _TB // 8, body, 0)
    plsc.subcore_barrier()
    pltpu.sync_copy(acc_s.at[pl.ds(r0, _RPT)],
                    out_hbm.at[cid, pl.ds(r0, _RPT)])


_deg_call = pl.kernel(
    _deg_body,
    out_type=jax.ShapeDtypeStruct((_NC, _NP, _D), jnp.float32),
    mesh=plsc.VectorSubcoreMesh(core_axis_name="c", subcore_axis_name="s"),
    scratch_types=[
        pltpu.VMEM_SHARED((_NP, _D), jnp.float32),
        pltpu.VMEM((_TB, _EB), jnp.int32),
        pltpu.VMEM((_ZB, _D), jnp.float32),
        pltpu.SemaphoreType.DMA,
    ],
)


def _mp_body(g_hbm, src_hbm, dst_hbm, zeros_hbm, out_hbm,
             acc_s, isrc_v, idst_v, buf0, buf1, gs0, gs1, ss0, ss1):
    cid = lax.axis_index("c")
    sid = lax.axis_index("s")
    r0 = sid * _RPT
    pltpu.sync_copy(zeros_hbm, buf0)
    for k in range(_RPT // _ZB):
        pltpu.sync_copy(buf0, acc_s.at[pl.ds(r0 + k * _ZB, _ZB)])
    plsc.subcore_barrier()

    def g(j, buf, sem):
        pltpu.async_copy(g_hbm.at[isrc_v.at[j]], buf, sem)

    def gwait(j, buf, sem):
        pltpu.make_async_copy(g_hbm.at[isrc_v.at[j]], buf, sem).wait()

    def s(j, buf, sem):
        pltpu.async_copy(buf, acc_s.at[idst_v.at[j]], sem, add=True)

    def swait(j, buf, sem):
        pltpu.make_async_copy(buf, acc_s.at[idst_v.at[j]], sem).wait()

    def run(base, nchunks):
        def chunk(c, carry):
            off = base + c * _TB_C
            pltpu.sync_copy(src_hbm.at[sid, pl.ds(off, _TB_C)], isrc_v)
            pltpu.sync_copy(dst_hbm.at[sid, pl.ds(off, _TB_C)], idst_v)
            g(0, buf0, gs0)
            g(1, buf1, gs1)

            def body(m, carry2):
                a2, b2, a, b = 2 * m - 2, 2 * m - 1, 2 * m, 2 * m + 1
                gwait(a2, buf0, gs0)
                s(a2, buf0, ss0)
                gwait(b2, buf1, gs1)
                s(b2, buf1, ss1)
                swait(a2, buf0, ss0)
                g(a, buf0, gs0)
                swait(b2, buf1, ss1)
                g(b, buf1, gs1)
                return carry2

            out = lax.fori_loop(1, _TB_C // 2, body, carry)
            gwait(_TB_C - 2, buf0, gs0)
            s(_TB_C - 2, buf0, ss0)
            gwait(_TB_C - 1, buf1, gs1)
            s(_TB_C - 1, buf1, ss1)
            swait(_TB_C - 2, buf0, ss0)
            swait(_TB_C - 1, buf1, ss1)
            return out

        lax.fori_loop(0, nchunks, chunk, 0)

    @pl.when(cid == 0)
    def _():
        run(0, _TB0 // _TB_C)

    @pl.when(cid == 1)
    def _():
        run(_TB0, _TB1 // _TB_C)

    plsc.subcore_barrier()
    pltpu.sync_copy(acc_s.at[pl.ds(r0, _RPT)],
                    out_hbm.at[cid, pl.ds(r0, _RPT)])


_mp_call = pl.kernel(
    _mp_body,
    out_type=jax.ShapeDtypeStruct((_NC, _NP, _D), jnp.float32),
    mesh=plsc.VectorSubcoreMesh(core_axis_name="c", subcore_axis_name="s"),
    scratch_types=[
        pltpu.VMEM_SHARED((_NP, _D), jnp.float32),
        pltpu.VMEM((_TB_C, _EB), jnp.int32),
        pltpu.VMEM((_TB_C, _EB), jnp.int32),
        pltpu.VMEM((_EB, _D), jnp.float32),
        pltpu.VMEM((_EB, _D), jnp.float32),
        pltpu.SemaphoreType.DMA,
        pltpu.SemaphoreType.DMA,
        pltpu.SemaphoreType.DMA,
        pltpu.SemaphoreType.DMA,
    ],
)


def _dis(dp_ref):
    return lax.rsqrt(1.0 + dp_ref[0, :, 0:1] + dp_ref[1, :, 0:1])


def _valid_rows():
    rows = pl.program_id(0) * _RB + lax.broadcasted_iota(jnp.int32, (_RB, 1), 0)
    return rows < _N


def _dense1(x_ref, w_ref, dp_ref, g_ref, s_ref):
    h = jnp.dot(x_ref[...], w_ref[...], preferred_element_type=jnp.float32)
    dis = _dis(dp_ref)
    g = jnp.where(_valid_rows(), h * dis, 0.0)
    g_ref[...] = g
    s_ref[...] = g * dis


def _dense2(a_ref, s_ref, dp_ref, b_ref, w_ref, g_ref, s2_ref):
    acc = a_ref[0] + a_ref[1]
    dis = _dis(dp_ref)
    pre = jnp.maximum(acc * dis + s_ref[...] + b_ref[...], 0.0)
    h2 = jnp.dot(pre, w_ref[...], preferred_element_type=jnp.float32)
    g = jnp.where(_valid_rows(), h2 * dis, 0.0)
    g_ref[...] = g
    s2_ref[...] = g * dis


def _dense3(a_ref, s_ref, dp_ref, b_ref, w_ref, b3_ref, o_ref):
    acc = a_ref[0] + a_ref[1]
    dis = _dis(dp_ref)
    pre = jnp.maximum(acc * dis + s_ref[...] + b_ref[...], 0.0)
    o_ref[...] = (jnp.dot(pre, w_ref[...], preferred_element_type=jnp.float32)
                  + b3_ref[...])


_spec_rows = pl.BlockSpec((_RB, _D), lambda i: (i, 0))
_spec_w = pl.BlockSpec((_D, _D), lambda i: (0, 0))
_spec_pair = pl.BlockSpec((_NC, _RB, _D), lambda i: (0, i, 0))
_spec_b = pl.BlockSpec((1, _D), lambda i: (0, 0))

_grid = _NP // _RB

_dense1_call = pl.pallas_call(
    _dense1,
    grid=(_grid,),
    in_specs=[_spec_rows, _spec_w, _spec_pair],
    out_specs=[_spec_rows, _spec_rows],
    out_shape=[jax.ShapeDtypeStruct((_NP, _D), jnp.float32),
               jax.ShapeDtypeStruct((_NP, _D), jnp.float32)],
)

_dense2_call = pl.pallas_call(
    _dense2,
    grid=(_grid,),
    in_specs=[_spec_pair, _spec_rows, _spec_pair, _spec_b, _spec_w],
    out_specs=[_spec_rows, _spec_rows],
    out_shape=[jax.ShapeDtypeStruct((_NP, _D), jnp.float32),
               jax.ShapeDtypeStruct((_NP, _D), jnp.float32)],
)

_dense3_call = pl.pallas_call(
    _dense3,
    grid=(_grid,),
    in_specs=[_spec_pair, _spec_rows, _spec_pair, _spec_b,
              pl.BlockSpec((_D, 1), lambda i: (0, 0)),
              pl.BlockSpec((1, 1), lambda i: (0, 0))],
    out_specs=pl.BlockSpec((_RB, 1), lambda i: (i, 0)),
    out_shape=jax.ShapeDtypeStruct((_NP, 1), jnp.float32),
)


@jax.jit
def kernel(x, edge_index, W1, b1, W2, b2, W3, b3):
    src = edge_index[0]
    dst = edge_index[1]
    pad = jnp.full((_EPAD - _E,), _N, jnp.int32)
    srcp = jnp.concatenate([src, pad]).reshape(_NS, _TB_PAIR, _EB)
    dstp = jnp.concatenate([dst, pad]).reshape(_NS, _TB_PAIR, _EB)
    x_pad = jnp.concatenate([x, jnp.zeros((_NP - _N, _D), x.dtype)])

    zo = jnp.stack([jnp.zeros((_ZB, _D), jnp.float32),
                    jnp.ones((_ZB, _D), jnp.float32)])
    zeros_pg = zo[0]

    degp = _deg_call(dstp, zo)

    g1, s1 = _dense1_call(x_pad, W1, degp)
    acc1 = _mp_call(g1, srcp, dstp, zeros_pg)
    g2, s2 = _dense2_call(acc1, s1, degp, b1.reshape(1, _D), W2)
    acc2 = _mp_call(g2, srcp, dstp, zeros_pg)
    out = _dense3_call(acc2, s2, degp, b2.reshape(1, _D), W3,
                       b3.reshape(1, 1))
    return out[:_N]


# speedup vs baseline: 2.4702x; 2.4702x over previous
import jax
import jax.numpy as jnp
from jax import lax
from jax.experimental import pallas as pl
from jax.experimental.pallas import tpu as pltpu
from jax.experimental.pallas import tpu_sc as plsc

_N = 10000
_E = 320000
_D = 128
_NC, _NS = 2, 16
_RPT = 640
_NP = _NS * _RPT
_EB = 128
_TB = 80
_TB_PAIR = 160
_TB0 = 80
_TB1 = _TB_PAIR - _TB0
_TB_C = 40
_EPAD = _NS * _TB_PAIR * _EB
_RB = 1024
_ZB = 128


def _deg_body(dst_hbm, ones_hbm, out_hbm, acc_s, idx_v, buf_v, sem):
    cid = lax.axis_index("c")
    sid = lax.axis_index("s")
    r0 = sid * _RPT
    pltpu.sync_copy(dst_hbm.at[sid, pl.ds(cid * _TB, _TB)], idx_v)
    pltpu.sync_copy(ones_hbm.at[0], buf_v)
    for k in range(_RPT // _ZB):
        pltpu.sync_copy(buf_v, acc_s.at[pl.ds(r0 + k * _ZB, _ZB)])
    pltpu.sync_copy(ones_hbm.at[1], buf_v)
    plsc.subcore_barrier()

    def body(m, carry):
        for k in range(8):
            pltpu.async_copy(buf_v, acc_s.at[idx_v.at[m * 8 + k]], sem,
                             add=True)
        for k in range(8):
            pltpu.make_async_copy(buf_v, acc_s.at[idx_v.at[m * 8 + k]],
                                  sem).wait()
        return carry

    lax.fori_loop(0, _TB // 8, body, 0)
    plsc.subcore_barrier()
    pltpu.sync_copy(acc_s.at[pl.ds(r0, _RPT)],
                    out_hbm.at[cid, pl.ds(r0, _RPT)])


_deg_call = pl.kernel(
    _deg_body,
    out_type=jax.ShapeDtypeStruct((_NC, _NP, _D), jnp.float32),
    mesh=plsc.VectorSubcoreMesh(core_axis_name="c", subcore_axis_name="s"),
    scratch_types=[
        pltpu.VMEM_SHARED((_NP, _D), jnp.float32),
        pltpu.VMEM((_TB, _EB), jnp.int32),
        pltpu.VMEM((_ZB, _D), jnp.float32),
        pltpu.SemaphoreType.DMA,
    ],
)


def _mp_body(g_hbm, src_hbm, dst_hbm, zeros_hbm, out_hbm,
             acc_s, isrc_v, idst_v, buf0, buf1, gs0, gs1, ss0, ss1):
    cid = lax.axis_index("c")
    sid = lax.axis_index("s")
    r0 = sid * _RPT
    pltpu.sync_copy(zeros_hbm, buf0)
    for k in range(_RPT // _ZB):
        pltpu.sync_copy(buf0, acc_s.at[pl.ds(r0 + k * _ZB, _ZB)])
    plsc.subcore_barrier()

    def g(j, buf, sem):
        pltpu.async_copy(g_hbm.at[isrc_v.at[j]], buf, sem)

    def gwait(j, buf, sem):
        pltpu.make_async_copy(g_hbm.at[isrc_v.at[j]], buf, sem).wait()

    def s(j, buf, sem):
        pltpu.async_copy(buf, acc_s.at[idst_v.at[j]], sem, add=True)

    def swait(j, buf, sem):
        pltpu.make_async_copy(buf, acc_s.at[idst_v.at[j]], sem).wait()

    def run(base, nchunks):
        def chunk(c, carry):
            off = base + c * _TB_C
            pltpu.sync_copy(src_hbm.at[sid, pl.ds(off, _TB_C)], isrc_v)
            pltpu.sync_copy(dst_hbm.at[sid, pl.ds(off, _TB_C)], idst_v)
            g(0, buf0, gs0)
            g(1, buf1, gs1)

            def body(m, carry2):
                a2, b2, a, b = 2 * m - 2, 2 * m - 1, 2 * m, 2 * m + 1
                gwait(a2, buf0, gs0)
                s(a2, buf0, ss0)
                gwait(b2, buf1, gs1)
                s(b2, buf1, ss1)
                swait(a2, buf0, ss0)
                g(a, buf0, gs0)
                swait(b2, buf1, ss1)
                g(b, buf1, gs1)
                return carry2

            out = lax.fori_loop(1, _TB_C // 2, body, carry)
            gwait(_TB_C - 2, buf0, gs0)
            s(_TB_C - 2, buf0, ss0)
            gwait(_TB_C - 1, buf1, gs1)
            s(_TB_C - 1, buf1, ss1)
            swait(_TB_C - 2, buf0, ss0)
            swait(_TB_C - 1, buf1, ss1)
            return out

        lax.fori_loop(0, nchunks, chunk, 0)

    @pl.when(cid == 0)
    def _():
        run(0, _TB0 // _TB_C)

    @pl.when(cid == 1)
    def _():
        run(_TB0, _TB1 // _TB_C)

    plsc.subcore_barrier()
    pltpu.sync_copy(acc_s.at[pl.ds(r0, _RPT)],
                    out_hbm.at[cid, pl.ds(r0, _RPT)])


_mp_call = pl.kernel(
    _mp_body,
    out_type=jax.ShapeDtypeStruct((_NC, _NP, _D), jnp.float32),
    mesh=plsc.VectorSubcoreMesh(core_axis_name="c", subcore_axis_name="s"),
    scratch_types=[
        pltpu.VMEM_SHARED((_NP, _D), jnp.float32),
        pltpu.VMEM((_TB_C, _EB), jnp.int32),
        pltpu.VMEM((_TB_C, _EB), jnp.int32),
        pltpu.VMEM((_EB, _D), jnp.float32),
        pltpu.VMEM((_EB, _D), jnp.float32),
        pltpu.SemaphoreType.DMA,
        pltpu.SemaphoreType.DMA,
        pltpu.SemaphoreType.DMA,
        pltpu.SemaphoreType.DMA,
    ],
)


def _dis(dp_ref):
    return lax.rsqrt(1.0 + dp_ref[0, :, 0:1] + dp_ref[1, :, 0:1])


def _valid_rows():
    rows = pl.program_id(0) * _RB + lax.broadcasted_iota(jnp.int32, (_RB, 1), 0)
    return rows < _N


def _dense1(x_ref, w_ref, dp_ref, g_ref, s_ref):
    h = jnp.dot(x_ref[...], w_ref[...], preferred_element_type=jnp.float32)
    dis = _dis(dp_ref)
    g = jnp.where(_valid_rows(), h * dis, 0.0)
    g_ref[...] = g
    s_ref[...] = g * dis


def _dense2(a_ref, s_ref, dp_ref, b_ref, w_ref, g_ref, s2_ref):
    acc = a_ref[0] + a_ref[1]
    dis = _dis(dp_ref)
    pre = jnp.maximum(acc * dis + s_ref[...] + b_ref[...], 0.0)
    h2 = jnp.dot(pre, w_ref[...], preferred_element_type=jnp.float32)
    g = jnp.where(_valid_rows(), h2 * dis, 0.0)
    g_ref[...] = g
    s2_ref[...] = g * dis


def _dense3(a_ref, s_ref, dp_ref, b_ref, w_ref, b3_ref, o_ref):
    acc = a_ref[0] + a_ref[1]
    dis = _dis(dp_ref)
    pre = jnp.maximum(acc * dis + s_ref[...] + b_ref[...], 0.0)
    o_ref[...] = (jnp.dot(pre, w_ref[...], preferred_element_type=jnp.float32)
                  + b3_ref[...])


_spec_rows = pl.BlockSpec((_RB, _D), lambda i: (i, 0))
_spec_w = pl.BlockSpec((_D, _D), lambda i: (0, 0))
_spec_pair = pl.BlockSpec((_NC, _RB, _D), lambda i: (0, i, 0))
_spec_b = pl.BlockSpec((1, _D), lambda i: (0, 0))

_grid = _NP // _RB

_dense1_call = pl.pallas_call(
    _dense1,
    grid=(_grid,),
    in_specs=[_spec_rows, _spec_w, _spec_pair],
    out_specs=[_spec_rows, _spec_rows],
    out_shape=[jax.ShapeDtypeStruct((_NP, _D), jnp.float32),
               jax.ShapeDtypeStruct((_NP, _D), jnp.float32)],
)

_dense2_call = pl.pallas_call(
    _dense2,
    grid=(_grid,),
    in_specs=[_spec_pair, _spec_rows, _spec_pair, _spec_b, _spec_w],
    out_specs=[_spec_rows, _spec_rows],
    out_shape=[jax.ShapeDtypeStruct((_NP, _D), jnp.float32),
               jax.ShapeDtypeStruct((_NP, _D), jnp.float32)],
)

_dense3_call = pl.pallas_call(
    _dense3,
    grid=(_grid,),
    in_specs=[_spec_pair, _spec_rows, _spec_pair, _spec_b,
              pl.BlockSpec((_D, 1), lambda i: (0, 0)),
              pl.BlockSpec((1, 1), lambda i: (0, 0))],
    out_specs=pl.BlockSpec((_RB, 1), lambda i: (i, 0)),
    out_shape=jax.ShapeDtypeStruct((_NP, 1), jnp.float32),
)


@jax.jit
def kernel(x, edge_index, W1, b1, W2, b2, W3, b3):
    src = edge_index[0]
    dst = edge_index[1]
    pad = (_N + jnp.arange(_EPAD - _E, dtype=jnp.int32) % (_NP - _N))
    srcp = jnp.concatenate([src, pad]).reshape(_NS, _TB_PAIR, _EB)
    dstp = jnp.concatenate([dst, pad]).reshape(_NS, _TB_PAIR, _EB)
    x_pad = jnp.concatenate([x, jnp.zeros((_NP - _N, _D), x.dtype)])

    zo = jnp.stack([jnp.zeros((_ZB, _D), jnp.float32),
                    jnp.ones((_ZB, _D), jnp.float32)])
    zeros_pg = zo[0]

    degp = _deg_call(dstp, zo)

    g1, s1 = _dense1_call(x_pad, W1, degp)
    acc1 = _mp_call(g1, srcp, dstp, zeros_pg)
    g2, s2 = _dense2_call(acc1, s1, degp, b1.reshape(1, _D), W2)
    acc2 = _mp_call(g2, srcp, dstp, zeros_pg)
    out = _dense3_call(acc2, s2, degp, b2.reshape(1, _D), W3,
                       b3.reshape(1, 1))
    return out[:_N]


# speedup vs baseline: 2.8269x; 1.1444x over previous
import jax
import jax.numpy as jnp
from jax import lax
from jax.experimental import pallas as pl
from jax.experimental.pallas import tpu as pltpu
from jax.experimental.pallas import tpu_sc as plsc

_N = 10000
_E = 320000
_D = 128
_NC, _NS = 2, 16
_RPT = 640
_NP = _NS * _RPT
_EB = 128
_TB = 80
_TB_PAIR = 160
_TB0 = 80
_TB1 = _TB_PAIR - _TB0
_TB_C = 40
_EPAD = _NS * _TB_PAIR * _EB
_RB = 1024
_ZB = 128


def _deg_body(dst_hbm, ones_hbm, out_hbm, acc_s, idx_v, buf_v, sem):
    cid = lax.axis_index("c")
    sid = lax.axis_index("s")
    r0 = sid * _RPT
    pltpu.sync_copy(dst_hbm.at[sid, pl.ds(cid * _TB, _TB)], idx_v)
    pltpu.sync_copy(ones_hbm.at[0], buf_v)
    for k in range(_RPT // _ZB):
        pltpu.sync_copy(buf_v, acc_s.at[pl.ds(r0 + k * _ZB, _ZB)])
    pltpu.sync_copy(ones_hbm.at[1], buf_v)
    plsc.subcore_barrier()

    def body(m, carry):
        for k in range(8):
            pltpu.async_copy(buf_v, acc_s.at[idx_v.at[m * 8 + k]], sem,
                             add=True)
        for k in range(8):
            pltpu.make_async_copy(buf_v, acc_s.at[idx_v.at[m * 8 + k]],
                                  sem).wait()
        return carry

    lax.fori_loop(0, _TB // 8, body, 0)
    plsc.subcore_barrier()
    pltpu.sync_copy(acc_s.at[pl.ds(r0, _RPT)],
                    out_hbm.at[cid, pl.ds(r0, _RPT)])


_deg_call = pl.kernel(
    _deg_body,
    out_type=jax.ShapeDtypeStruct((_NC, _NP, _D), jnp.float32),
    mesh=plsc.VectorSubcoreMesh(core_axis_name="c", subcore_axis_name="s"),
    scratch_types=[
        pltpu.VMEM_SHARED((_NP, _D), jnp.float32),
        pltpu.VMEM((_TB, _EB), jnp.int32),
        pltpu.VMEM((_ZB, _D), jnp.float32),
        pltpu.SemaphoreType.DMA,
    ],
)


def _mp_body(g_hbm, src_hbm, dst_hbm, zeros_hbm, out_hbm,
             acc_s, isrc_v, idst_v, buf0, buf1,
             gs0, gs1, gs2, gs3, ss0, ss1, ss2, ss3):
    cid = lax.axis_index("c")
    sid = lax.axis_index("s")
    r0 = sid * _RPT
    pltpu.sync_copy(zeros_hbm, buf0)
    for k in range(_RPT // _ZB):
        pltpu.sync_copy(buf0, acc_s.at[pl.ds(r0 + k * _ZB, _ZB)])
    plsc.subcore_barrier()

    half = _EB // 2
    bufs = [[buf0.at[pl.ds(0, half)], buf0.at[pl.ds(half, half)]],
            [buf1.at[pl.ds(0, half)], buf1.at[pl.ds(half, half)]]]
    gsem = [[gs0, gs1], [gs2, gs3]]
    ssem = [[ss0, ss1], [ss2, ss3]]

    def isrc(j, h):
        return isrc_v.at[j, pl.ds(h * half, half)]

    def idst(j, h):
        return idst_v.at[j, pl.ds(h * half, half)]

    def g(j, bi):
        for h in range(2):
            pltpu.async_copy(g_hbm.at[isrc(j, h)], bufs[bi][h], gsem[bi][h])

    def gwait(j, bi):
        for h in range(2):
            pltpu.make_async_copy(g_hbm.at[isrc(j, h)], bufs[bi][h],
                                  gsem[bi][h]).wait()

    def s(j, bi):
        for h in range(2):
            pltpu.async_copy(bufs[bi][h], acc_s.at[idst(j, h)], ssem[bi][h],
                             add=True)

    def swait(j, bi):
        for h in range(2):
            pltpu.make_async_copy(bufs[bi][h], acc_s.at[idst(j, h)],
                                  ssem[bi][h]).wait()

    def run(base, nchunks):
        def chunk(c, carry):
            off = base + c * _TB_C
            pltpu.sync_copy(src_hbm.at[sid, pl.ds(off, _TB_C)], isrc_v)
            pltpu.sync_copy(dst_hbm.at[sid, pl.ds(off, _TB_C)], idst_v)
            g(0, 0)
            g(1, 1)

            def body(m, carry2):
                a2, b2, a, b = 2 * m - 2, 2 * m - 1, 2 * m, 2 * m + 1
                gwait(a2, 0)
                s(a2, 0)
                gwait(b2, 1)
                s(b2, 1)
                swait(a2, 0)
                g(a, 0)
                swait(b2, 1)
                g(b, 1)
                return carry2

            out = lax.fori_loop(1, _TB_C // 2, body, carry)
            gwait(_TB_C - 2, 0)
            s(_TB_C - 2, 0)
            gwait(_TB_C - 1, 1)
            s(_TB_C - 1, 1)
            swait(_TB_C - 2, 0)
            swait(_TB_C - 1, 1)
            return out

        lax.fori_loop(0, nchunks, chunk, 0)

    @pl.when(cid == 0)
    def _():
        run(0, _TB0 // _TB_C)

    @pl.when(cid == 1)
    def _():
        run(_TB0, _TB1 // _TB_C)

    plsc.subcore_barrier()
    pltpu.sync_copy(acc_s.at[pl.ds(r0, _RPT)],
                    out_hbm.at[cid, pl.ds(r0, _RPT)])


_mp_call = pl.kernel(
    _mp_body,
    out_type=jax.ShapeDtypeStruct((_NC, _NP, _D), jnp.float32),
    mesh=plsc.VectorSubcoreMesh(core_axis_name="c", subcore_axis_name="s"),
    scratch_types=[
        pltpu.VMEM_SHARED((_NP, _D), jnp.float32),
        pltpu.VMEM((_TB_C, _EB), jnp.int32),
        pltpu.VMEM((_TB_C, _EB), jnp.int32),
        pltpu.VMEM((_EB, _D), jnp.float32),
        pltpu.VMEM((_EB, _D), jnp.float32),
        pltpu.SemaphoreType.DMA,
        pltpu.SemaphoreType.DMA,
        pltpu.SemaphoreType.DMA,
        pltpu.SemaphoreType.DMA,
        pltpu.SemaphoreType.DMA,
        pltpu.SemaphoreType.DMA,
        pltpu.SemaphoreType.DMA,
        pltpu.SemaphoreType.DMA,
    ],
)


def _dis(dp_ref):
    return lax.rsqrt(1.0 + dp_ref[0, :, 0:1] + dp_ref[1, :, 0:1])


def _valid_rows():
    rows = pl.program_id(0) * _RB + lax.broadcasted_iota(jnp.int32, (_RB, 1), 0)
    return rows < _N


def _dense1(x_ref, w_ref, dp_ref, g_ref, s_ref):
    h = jnp.dot(x_ref[...], w_ref[...], preferred_element_type=jnp.float32)
    dis = _dis(dp_ref)
    g = jnp.where(_valid_rows(), h * dis, 0.0)
    g_ref[...] = g
    s_ref[...] = g * dis


def _dense2(a_ref, s_ref, dp_ref, b_ref, w_ref, g_ref, s2_ref):
    acc = a_ref[0] + a_ref[1]
    dis = _dis(dp_ref)
    pre = jnp.maximum(acc * dis + s_ref[...] + b_ref[...], 0.0)
    h2 = jnp.dot(pre, w_ref[...], preferred_element_type=jnp.float32)
    g = jnp.where(_valid_rows(), h2 * dis, 0.0)
    g_ref[...] = g
    s2_ref[...] = g * dis


def _dense3(a_ref, s_ref, dp_ref, b_ref, w_ref, b3_ref, o_ref):
    acc = a_ref[0] + a_ref[1]
    dis = _dis(dp_ref)
    pre = jnp.maximum(acc * dis + s_ref[...] + b_ref[...], 0.0)
    o_ref[...] = (jnp.dot(pre, w_ref[...], preferred_element_type=jnp.float32)
                  + b3_ref[...])


_spec_rows = pl.BlockSpec((_RB, _D), lambda i: (i, 0))
_spec_w = pl.BlockSpec((_D, _D), lambda i: (0, 0))
_spec_pair = pl.BlockSpec((_NC, _RB, _D), lambda i: (0, i, 0))
_spec_b = pl.BlockSpec((1, _D), lambda i: (0, 0))

_grid = _NP // _RB

_dense1_call = pl.pallas_call(
    _dense1,
    grid=(_grid,),
    in_specs=[_spec_rows, _spec_w, _spec_pair],
    out_specs=[_spec_rows, _spec_rows],
    out_shape=[jax.ShapeDtypeStruct((_NP, _D), jnp.float32),
               jax.ShapeDtypeStruct((_NP, _D), jnp.float32)],
)

_dense2_call = pl.pallas_call(
    _dense2,
    grid=(_grid,),
    in_specs=[_spec_pair, _spec_rows, _spec_pair, _spec_b, _spec_w],
    out_specs=[_spec_rows, _spec_rows],
    out_shape=[jax.ShapeDtypeStruct((_NP, _D), jnp.float32),
               jax.ShapeDtypeStruct((_NP, _D), jnp.float32)],
)

_dense3_call = pl.pallas_call(
    _dense3,
    grid=(_grid,),
    in_specs=[_spec_pair, _spec_rows, _spec_pair, _spec_b,
              pl.BlockSpec((_D, 1), lambda i: (0, 0)),
              pl.BlockSpec((1, 1), lambda i: (0, 0))],
    out_specs=pl.BlockSpec((_RB, 1), lambda i: (i, 0)),
    out_shape=jax.ShapeDtypeStruct((_NP, 1), jnp.float32),
)


@jax.jit
def kernel(x, edge_index, W1, b1, W2, b2, W3, b3):
    src = edge_index[0]
    dst = edge_index[1]
    pad = (_N + jnp.arange(_EPAD - _E, dtype=jnp.int32) % (_NP - _N))
    srcp = jnp.concatenate([src, pad]).reshape(_NS, _TB_PAIR, _EB)
    dstp = jnp.concatenate([dst, pad]).reshape(_NS, _TB_PAIR, _EB)
    x_pad = jnp.concatenate([x, jnp.zeros((_NP - _N, _D), x.dtype)])

    zo = jnp.stack([jnp.zeros((_ZB, _D), jnp.float32),
                    jnp.ones((_ZB, _D), jnp.float32)])
    zeros_pg = zo[0]

    degp = _deg_call(dstp, zo)

    g1, s1 = _dense1_call(x_pad, W1, degp)
    acc1 = _mp_call(g1, srcp, dstp, zeros_pg)
    g2, s2 = _dense2_call(acc1, s1, degp, b1.reshape(1, _D), W2)
    acc2 = _mp_call(g2, srcp, dstp, zeros_pg)
    out = _dense3_call(acc2, s2, degp, b2.reshape(1, _D), W3,
                       b3.reshape(1, 1))
    return out[:_N]
